# baseline (speedup 1.0000x reference)
import functools
import math

import jax
import jax.numpy as jnp
from jax import lax
from jax.experimental import pallas as pl
from jax.experimental.pallas import tpu as pltpu
from jax.experimental.pallas import tpu_sc as plsc

_CHUNK = 256
_NBUF = 3


def _pe_table(L, E, rows):
  def body(o_ref):
    r = lax.broadcasted_iota(jnp.int32, (rows, 128), 0)
    c = lax.broadcasted_iota(jnp.int32, (rows, 128), 1)
    pos = (r % L).astype(jnp.float32) + 1.0
    expnt = (2 * (c >> 1)).astype(jnp.float32) * (math.log(10000.0) / E)
    ang = pos * jnp.exp(-expnt)
    pe = jnp.where(c % 2 == 0, jnp.cos(ang), jnp.sin(ang))
    o_ref[...] = jnp.where(c < E, pe, 0.0)

  return pl.pallas_call(
      body, out_shape=jax.ShapeDtypeStruct((rows, 128), jnp.float32))()


@functools.cache
def _make_emb(B, L, E):
  info = plsc.get_sparse_core_info()
  NC, NS = info.num_cores, info.num_subcores
  NW = NC * NS
  T = B * L
  assert T % (NW * _CHUNK) == 0 and E == 64
  tok_per_w = T // NW
  n_chunks = tok_per_w // _CHUNK
  pe_rows = L + _CHUNK
  mesh = plsc.VectorSubcoreMesh(core_axis_name="c", subcore_axis_name="s")

  @functools.partial(
      pl.kernel,
      out_type=jax.ShapeDtypeStruct((T * E // 128, 128), jnp.float32),
      mesh=mesh,
      scratch_types=[
          pltpu.VMEM((_NBUF, _CHUNK), jnp.int32),
          pltpu.VMEM((_NBUF, _CHUNK, 128), jnp.float32),
          pltpu.VMEM_SHARED((pe_rows, 128), jnp.float32),
          pltpu.SemaphoreType.DMA,
          pltpu.SemaphoreType.DMA,
      ],
  )
  def emb(x_hbm, w_hbm, pe_hbm, out_hbm, idx_v, rows_v, pe_sh, gsem, osem):
    sid = lax.axis_index("s")
    wid = sid * NC + lax.axis_index("c")
    tok0 = wid * tok_per_w

    @pl.when(sid == 0)
    def _fill_pe():
      pltpu.sync_copy(pe_hbm, pe_sh)

    plsc.subcore_barrier()

    def issue(k, buf):
      start = tok0 + k * _CHUNK
      pltpu.sync_copy(x_hbm.at[pl.ds(start, _CHUNK)], idx_v.at[buf])
      off = lax.rem(start, L)
      pltpu.sync_copy(pe_sh.at[pl.ds(off, _CHUNK)], rows_v.at[buf])
      for j in range(_CHUNK // 128):
        pltpu.async_copy(
            w_hbm.at[idx_v.at[buf].at[pl.ds(j * 128, 128)]],
            rows_v.at[buf].at[pl.ds(j * 128, 128)], gsem, add=True)

    def drain_gather(buf):
      for j in range(_CHUNK // 128):
        pltpu.make_async_copy(
            w_hbm.at[pl.ds(0, 128)],
            rows_v.at[buf].at[pl.ds(j * 128, 128)], gsem).wait()

    issue(0, 0)

    def chunk_body(k, carry):
      buf = lax.rem(k, _NBUF)

      @pl.when(k >= _NBUF - 1)
      def _drain_out():
        pltpu.make_async_copy(
            rows_v.at[0].at[pl.ds(0, _CHUNK // 2)],
            out_hbm.at[pl.ds(0, _CHUNK // 2)], osem).wait()

      @pl.when(k + 1 < n_chunks)
      def _issue_next():
        issue(k + 1, lax.rem(k + 1, _NBUF))

      drain_gather(buf)

      def r_body(r, c):
        for half in range(2):
          for jj in range(E // 16):
            rows_v[buf, r, pl.ds(half * E + jj * 16, 16)] = (
                rows_v[buf, 2 * r + half, pl.ds(jj * 16, 16)])
        return c

      lax.fori_loop(0, _CHUNK // 2, r_body, 0)

      orow = pl.multiple_of((tok0 + k * _CHUNK) * E // 128, _CHUNK // 2)
      pltpu.async_copy(
          rows_v.at[buf].at[pl.ds(0, _CHUNK // 2)],
          out_hbm.at[pl.ds(orow, _CHUNK // 2)], osem)
      return carry

    lax.fori_loop(0, n_chunks, chunk_body, 0)
    for _ in range(2):
      pltpu.make_async_copy(
          rows_v.at[0].at[pl.ds(0, _CHUNK // 2)],
          out_hbm.at[pl.ds(0, _CHUNK // 2)], osem).wait()

  return emb


def kernel(x_batch, W):
  B, L = x_batch.shape
  _, E = W.shape
  pe = _pe_table(L, E, L + _CHUNK)
  x = x_batch.astype(jnp.int32).reshape(B * L)
  W128 = jnp.pad(W, ((0, 0), (0, 128 - E)))
  out = _make_emb(B, L, E)(x, W128, pe)
  return out.reshape(B, L, E)
